# baseline (speedup 1.0000x reference)
import functools

import jax
import jax.numpy as jnp
from jax import lax
from jax.experimental import pallas as pl
from jax.experimental.pallas import tpu as pltpu
from jax.experimental.pallas import tpu_sc as plsc

N_NODES = 10000
N_EDGES = 320000
D_IN = 128
D_OUT = 128

NC = 2
NS = 16
NW = NC * NS
EPW = N_EDGES // NW
CHUNK = 80
NCHUNK = EPW // CHUNK
ROWS_PT = 624
ROWS_LAST = N_NODES - (NS - 1) * ROWS_PT
LANES = 16
VPR = D_OUT // LANES


def _mm_body(x_ref, w_ref, o_ref):
    o_ref[...] = jnp.dot(x_ref[...], w_ref[...],
                         preferred_element_type=jnp.float32)


def _matmul(x, W):
    grid = 10
    rows = N_NODES // grid
    return pl.pallas_call(
        _mm_body,
        grid=(grid,),
        in_specs=[
            pl.BlockSpec((rows, D_IN), lambda i: (i, 0)),
            pl.BlockSpec((D_IN, D_OUT), lambda i: (0, 0)),
        ],
        out_specs=pl.BlockSpec((rows, D_OUT), lambda i: (i, 0)),
        out_shape=jax.ShapeDtypeStruct((N_NODES, D_OUT), jnp.float32),
    )(x, W)


_mesh = plsc.VectorSubcoreMesh(core_axis_name="c", subcore_axis_name="s")


@functools.partial(
    pl.kernel,
    out_type=jax.ShapeDtypeStruct((NC, N_NODES, D_OUT), jnp.float32),
    mesh=_mesh,
    scratch_types=[
        pltpu.VMEM((CHUNK,), jnp.int32),
        pltpu.VMEM((CHUNK,), jnp.int32),
        pltpu.VMEM((CHUNK,), jnp.float32),
        pltpu.VMEM((CHUNK,), jnp.int32),
        pltpu.VMEM((CHUNK,), jnp.int32),
        pltpu.VMEM((CHUNK,), jnp.float32),
        pltpu.VMEM((CHUNK,), jnp.int32),
        pltpu.VMEM((CHUNK,), jnp.int32),
        pltpu.VMEM((CHUNK,), jnp.float32),
        pltpu.VMEM((CHUNK, D_OUT), jnp.float32),
        pltpu.VMEM((CHUNK, D_OUT), jnp.float32),
        pltpu.VMEM((CHUNK, D_OUT), jnp.float32),
        pltpu.VMEM_SHARED((N_NODES, D_OUT), jnp.float32),
        pltpu.SemaphoreType.DMA,
        pltpu.SemaphoreType.DMA,
        pltpu.SemaphoreType.DMA,
        pltpu.SemaphoreType.DMA,
        pltpu.SemaphoreType.DMA,
        pltpu.SemaphoreType.DMA,
    ],
)
def _sc_edges(src_hbm, dst_hbm, w_hbm, h_hbm, out_hbm,
              src_vA, dst_vA, w_vA, src_vB, dst_vB, w_vB,
              src_vC, dst_vC, w_vC,
              rows_vA, rows_vB, rows_vC, acc_sh,
              gatA, gatB, gatC, scatA, scatB, scatC):
    cid = lax.axis_index("c")
    sid = lax.axis_index("s")
    wid = sid * NC + cid

    zvec = jnp.zeros((LANES,), jnp.float32)

    def _zero_row(r, _):
        for j in range(VPR):
            rows_vA[r, pl.ds(j * LANES, LANES)] = zvec
        return 0

    lax.fori_loop(0, CHUNK, _zero_row, 0)
    row_base = pl.multiple_of(sid * ROWS_PT, 8)
    for z in range(ROWS_PT // CHUNK):
        pltpu.sync_copy(rows_vA,
                        acc_sh.at[pl.ds(row_base + z * CHUNK, CHUNK)])

    @pl.when(sid < NS - 1)
    def _zero_tail():
        pltpu.sync_copy(
            rows_vA.at[pl.ds(0, ROWS_PT - 7 * CHUNK)],
            acc_sh.at[pl.ds(row_base + 7 * CHUNK, ROWS_PT - 7 * CHUNK)])

    @pl.when(sid == NS - 1)
    def _zero_tail_last():
        pltpu.sync_copy(
            rows_vA,
            acc_sh.at[pl.ds((NS - 1) * ROWS_PT + 7 * CHUNK, CHUNK)])

    plsc.subcore_barrier()

    def _scale(w_v, rows_v):
        def _group(g, _):
            wv = w_v[pl.ds(g * LANES, LANES)]
            for t in range(LANES):
                e = g * LANES + t
                w = wv[t]
                for j in range(VPR):
                    sl = pl.ds(j * LANES, LANES)
                    rows_v[e, sl] = rows_v[e, sl] * w
            return 0

        lax.fori_loop(0, CHUNK // LANES, _group, 0)

    def _triple(t, _):
        base0 = pl.multiple_of(wid * EPW + (3 * t) * CHUNK, CHUNK)
        base1 = pl.multiple_of(wid * EPW + (3 * t + 1) * CHUNK, CHUNK)
        base2 = pl.multiple_of(wid * EPW + (3 * t + 2) * CHUNK, CHUNK)
        pltpu.sync_copy(src_hbm.at[pl.ds(base0, CHUNK)], src_vA)
        pltpu.sync_copy(dst_hbm.at[pl.ds(base0, CHUNK)], dst_vA)
        pltpu.sync_copy(w_hbm.at[pl.ds(base0, CHUNK)], w_vA)
        g0 = pltpu.async_copy(h_hbm.at[src_vA], rows_vA, gatA)
        pltpu.sync_copy(src_hbm.at[pl.ds(base1, CHUNK)], src_vB)
        pltpu.sync_copy(dst_hbm.at[pl.ds(base1, CHUNK)], dst_vB)
        pltpu.sync_copy(w_hbm.at[pl.ds(base1, CHUNK)], w_vB)
        g0.wait()
        g1 = pltpu.async_copy(h_hbm.at[src_vB], rows_vB, gatB)
        pltpu.sync_copy(src_hbm.at[pl.ds(base2, CHUNK)], src_vC)
        pltpu.sync_copy(dst_hbm.at[pl.ds(base2, CHUNK)], dst_vC)
        pltpu.sync_copy(w_hbm.at[pl.ds(base2, CHUNK)], w_vC)
        _scale(w_vA, rows_vA)
        s0 = pltpu.async_copy(rows_vA, acc_sh.at[dst_vA], scatA, add=True)
        g1.wait()
        g2 = pltpu.async_copy(h_hbm.at[src_vC], rows_vC, gatC)
        _scale(w_vB, rows_vB)
        s1 = pltpu.async_copy(rows_vB, acc_sh.at[dst_vB], scatB, add=True)
        g2.wait()
        _scale(w_vC, rows_vC)
        s0.wait()
        s1.wait()
        s2 = pltpu.async_copy(rows_vC, acc_sh.at[dst_vC], scatC, add=True)
        s2.wait()
        return 0

    lax.fori_loop(0, NCHUNK // 3, _triple, 0)

    baseT0 = pl.multiple_of(wid * EPW + (NCHUNK - 2) * CHUNK, CHUNK)
    baseT1 = pl.multiple_of(wid * EPW + (NCHUNK - 1) * CHUNK, CHUNK)
    pltpu.sync_copy(src_hbm.at[pl.ds(baseT0, CHUNK)], src_vA)
    pltpu.sync_copy(dst_hbm.at[pl.ds(baseT0, CHUNK)], dst_vA)
    pltpu.sync_copy(w_hbm.at[pl.ds(baseT0, CHUNK)], w_vA)
    gT0 = pltpu.async_copy(h_hbm.at[src_vA], rows_vA, gatA)
    pltpu.sync_copy(src_hbm.at[pl.ds(baseT1, CHUNK)], src_vB)
    pltpu.sync_copy(dst_hbm.at[pl.ds(baseT1, CHUNK)], dst_vB)
    pltpu.sync_copy(w_hbm.at[pl.ds(baseT1, CHUNK)], w_vB)
    gT0.wait()
    gT1 = pltpu.async_copy(h_hbm.at[src_vB], rows_vB, gatB)
    _scale(w_vA, rows_vA)
    sT0 = pltpu.async_copy(rows_vA, acc_sh.at[dst_vA], scatA, add=True)
    gT1.wait()
    _scale(w_vB, rows_vB)
    sT0.wait()
    sT1 = pltpu.async_copy(rows_vB, acc_sh.at[dst_vB], scatB, add=True)
    sT1.wait()
    plsc.subcore_barrier()

    @pl.when(sid < NS - 1)
    def _wb_main():
        pltpu.sync_copy(acc_sh.at[pl.ds(row_base, ROWS_PT)],
                        out_hbm.at[cid, pl.ds(row_base, ROWS_PT)])

    @pl.when(sid == NS - 1)
    def _wb_last():
        last = (NS - 1) * ROWS_PT
        pltpu.sync_copy(acc_sh.at[pl.ds(last, ROWS_LAST)],
                        out_hbm.at[cid, pl.ds(last, ROWS_LAST)])


def _add_body(a_ref, b_ref, o_ref):
    o_ref[...] = a_ref[...] + b_ref[...]


def _combine(p0, p1):
    grid = 10
    rows = N_NODES // grid
    return pl.pallas_call(
        _add_body,
        grid=(grid,),
        in_specs=[
            pl.BlockSpec((rows, D_OUT), lambda i: (i, 0)),
            pl.BlockSpec((rows, D_OUT), lambda i: (i, 0)),
        ],
        out_specs=pl.BlockSpec((rows, D_OUT), lambda i: (i, 0)),
        out_shape=jax.ShapeDtypeStruct((N_NODES, D_OUT), jnp.float32),
    )(p0, p1)


def kernel(x, edge_index, edge_weight, W):
    edge_index = edge_index.astype(jnp.int32)
    src = edge_index[0]
    dst = edge_index[1]
    h = _matmul(x, W)
    partials = _sc_edges(src, dst, edge_weight, h)
    return _combine(partials[0], partials[1])


# speedup vs baseline: 1.0019x; 1.0019x over previous
import functools

import jax
import jax.numpy as jnp
from jax import lax
from jax.experimental import pallas as pl
from jax.experimental.pallas import tpu as pltpu
from jax.experimental.pallas import tpu_sc as plsc

N_NODES = 10000
N_EDGES = 320000
D_IN = 128
D_OUT = 128

NC = 2
NS = 16
NW = NC * NS
EPW = N_EDGES // NW
CHUNK = 80
NCHUNK = EPW // CHUNK
ROWS_PT = 624
ROWS_LAST = N_NODES - (NS - 1) * ROWS_PT
LANES = 16
VPR = D_OUT // LANES


def _mm_body(x_ref, w_ref, o_ref):
    o_ref[...] = jnp.dot(x_ref[...], w_ref[...],
                         preferred_element_type=jnp.float32)


def _matmul(x, W):
    grid = 10
    rows = N_NODES // grid
    return pl.pallas_call(
        _mm_body,
        grid=(grid,),
        in_specs=[
            pl.BlockSpec((rows, D_IN), lambda i: (i, 0)),
            pl.BlockSpec((D_IN, D_OUT), lambda i: (0, 0)),
        ],
        out_specs=pl.BlockSpec((rows, D_OUT), lambda i: (i, 0)),
        out_shape=jax.ShapeDtypeStruct((N_NODES, D_OUT), jnp.float32),
    )(x, W)


_mesh = plsc.VectorSubcoreMesh(core_axis_name="c", subcore_axis_name="s")


@functools.partial(
    pl.kernel,
    out_type=jax.ShapeDtypeStruct((NC, N_NODES, D_OUT), jnp.float32),
    mesh=_mesh,
    scratch_types=[
        pltpu.VMEM((CHUNK,), jnp.int32),
        pltpu.VMEM((CHUNK,), jnp.int32),
        pltpu.VMEM((CHUNK,), jnp.float32),
        pltpu.VMEM((CHUNK,), jnp.int32),
        pltpu.VMEM((CHUNK,), jnp.int32),
        pltpu.VMEM((CHUNK,), jnp.float32),
        pltpu.VMEM((CHUNK,), jnp.int32),
        pltpu.VMEM((CHUNK,), jnp.int32),
        pltpu.VMEM((CHUNK,), jnp.float32),
        pltpu.VMEM((CHUNK, D_OUT), jnp.float32),
        pltpu.VMEM((CHUNK, D_OUT), jnp.float32),
        pltpu.VMEM((CHUNK, D_OUT), jnp.float32),
        pltpu.VMEM_SHARED((N_NODES, D_OUT), jnp.float32),
        pltpu.SemaphoreType.DMA,
        pltpu.SemaphoreType.DMA,
        pltpu.SemaphoreType.DMA,
        pltpu.SemaphoreType.DMA,
        pltpu.SemaphoreType.DMA,
        pltpu.SemaphoreType.DMA,
    ],
)
def _sc_edges(src_hbm, dst_hbm, w_hbm, h_hbm, out_hbm,
              src_vA, dst_vA, w_vA, src_vB, dst_vB, w_vB,
              src_vC, dst_vC, w_vC,
              rows_vA, rows_vB, rows_vC, acc_sh,
              gatA, gatB, gatC, scatA, scatB, scatC):
    cid = lax.axis_index("c")
    sid = lax.axis_index("s")
    wid = sid * NC + cid

    zvec = jnp.zeros((LANES,), jnp.float32)

    def _zero_row(r, _):
        for j in range(VPR):
            rows_vA[r, pl.ds(j * LANES, LANES)] = zvec
        return 0

    lax.fori_loop(0, CHUNK, _zero_row, 0)
    row_base = pl.multiple_of(sid * ROWS_PT, 8)
    for z in range(ROWS_PT // CHUNK):
        pltpu.sync_copy(rows_vA,
                        acc_sh.at[pl.ds(row_base + z * CHUNK, CHUNK)])

    @pl.when(sid < NS - 1)
    def _zero_tail():
        pltpu.sync_copy(
            rows_vA.at[pl.ds(0, ROWS_PT - 7 * CHUNK)],
            acc_sh.at[pl.ds(row_base + 7 * CHUNK, ROWS_PT - 7 * CHUNK)])

    @pl.when(sid == NS - 1)
    def _zero_tail_last():
        pltpu.sync_copy(
            rows_vA,
            acc_sh.at[pl.ds((NS - 1) * ROWS_PT + 7 * CHUNK, CHUNK)])

    plsc.subcore_barrier()

    def _scale(w_v, rows_v):
        def _group(g, _):
            wv = w_v[pl.ds(g * LANES, LANES)]
            for t in range(LANES):
                e = g * LANES + t
                w = wv[t]
                for j in range(VPR):
                    sl = pl.ds(j * LANES, LANES)
                    rows_v[e, sl] = rows_v[e, sl] * w
            return 0

        lax.fori_loop(0, CHUNK // LANES, _group, 0)

    def _triple(t, _):
        base0 = pl.multiple_of(wid * EPW + (3 * t) * CHUNK, CHUNK)
        base1 = pl.multiple_of(wid * EPW + (3 * t + 1) * CHUNK, CHUNK)
        base2 = pl.multiple_of(wid * EPW + (3 * t + 2) * CHUNK, CHUNK)
        pltpu.sync_copy(src_hbm.at[pl.ds(base0, CHUNK)], src_vA)
        pltpu.sync_copy(dst_hbm.at[pl.ds(base0, CHUNK)], dst_vA)
        pltpu.sync_copy(w_hbm.at[pl.ds(base0, CHUNK)], w_vA)
        g0 = pltpu.async_copy(h_hbm.at[src_vA], rows_vA, gatA)
        pltpu.sync_copy(src_hbm.at[pl.ds(base1, CHUNK)], src_vB)
        pltpu.sync_copy(dst_hbm.at[pl.ds(base1, CHUNK)], dst_vB)
        pltpu.sync_copy(w_hbm.at[pl.ds(base1, CHUNK)], w_vB)
        g0.wait()
        g1 = pltpu.async_copy(h_hbm.at[src_vB], rows_vB, gatB)
        pltpu.sync_copy(src_hbm.at[pl.ds(base2, CHUNK)], src_vC)
        pltpu.sync_copy(dst_hbm.at[pl.ds(base2, CHUNK)], dst_vC)
        pltpu.sync_copy(w_hbm.at[pl.ds(base2, CHUNK)], w_vC)
        _scale(w_vA, rows_vA)
        s0 = pltpu.async_copy(rows_vA, acc_sh.at[dst_vA], scatA, add=True)
        g1.wait()
        g2 = pltpu.async_copy(h_hbm.at[src_vC], rows_vC, gatC)
        _scale(w_vB, rows_vB)
        s0.wait()
        s1 = pltpu.async_copy(rows_vB, acc_sh.at[dst_vB], scatB, add=True)
        g2.wait()
        _scale(w_vC, rows_vC)
        s1.wait()
        s2 = pltpu.async_copy(rows_vC, acc_sh.at[dst_vC], scatC, add=True)
        s2.wait()
        return 0

    lax.fori_loop(0, NCHUNK // 3, _triple, 0)

    baseT0 = pl.multiple_of(wid * EPW + (NCHUNK - 2) * CHUNK, CHUNK)
    baseT1 = pl.multiple_of(wid * EPW + (NCHUNK - 1) * CHUNK, CHUNK)
    pltpu.sync_copy(src_hbm.at[pl.ds(baseT0, CHUNK)], src_vA)
    pltpu.sync_copy(dst_hbm.at[pl.ds(baseT0, CHUNK)], dst_vA)
    pltpu.sync_copy(w_hbm.at[pl.ds(baseT0, CHUNK)], w_vA)
    gT0 = pltpu.async_copy(h_hbm.at[src_vA], rows_vA, gatA)
    pltpu.sync_copy(src_hbm.at[pl.ds(baseT1, CHUNK)], src_vB)
    pltpu.sync_copy(dst_hbm.at[pl.ds(baseT1, CHUNK)], dst_vB)
    pltpu.sync_copy(w_hbm.at[pl.ds(baseT1, CHUNK)], w_vB)
    gT0.wait()
    gT1 = pltpu.async_copy(h_hbm.at[src_vB], rows_vB, gatB)
    _scale(w_vA, rows_vA)
    sT0 = pltpu.async_copy(rows_vA, acc_sh.at[dst_vA], scatA, add=True)
    gT1.wait()
    _scale(w_vB, rows_vB)
    sT0.wait()
    sT1 = pltpu.async_copy(rows_vB, acc_sh.at[dst_vB], scatB, add=True)
    sT1.wait()
    plsc.subcore_barrier()

    @pl.when(sid < NS - 1)
    def _wb_main():
        pltpu.sync_copy(acc_sh.at[pl.ds(row_base, ROWS_PT)],
                        out_hbm.at[cid, pl.ds(row_base, ROWS_PT)])

    @pl.when(sid == NS - 1)
    def _wb_last():
        last = (NS - 1) * ROWS_PT
        pltpu.sync_copy(acc_sh.at[pl.ds(last, ROWS_LAST)],
                        out_hbm.at[cid, pl.ds(last, ROWS_LAST)])


def _add_body(a_ref, b_ref, o_ref):
    o_ref[...] = a_ref[...] + b_ref[...]


def _combine(p0, p1):
    grid = 10
    rows = N_NODES // grid
    return pl.pallas_call(
        _add_body,
        grid=(grid,),
        in_specs=[
            pl.BlockSpec((rows, D_OUT), lambda i: (i, 0)),
            pl.BlockSpec((rows, D_OUT), lambda i: (i, 0)),
        ],
        out_specs=pl.BlockSpec((rows, D_OUT), lambda i: (i, 0)),
        out_shape=jax.ShapeDtypeStruct((N_NODES, D_OUT), jnp.float32),
    )(p0, p1)


def kernel(x, edge_index, edge_weight, W):
    edge_index = edge_index.astype(jnp.int32)
    src = edge_index[0]
    dst = edge_index[1]
    h = _matmul(x, W)
    partials = _sc_edges(src, dst, edge_weight, h)
    return _combine(partials[0], partials[1])
